# baseline (speedup 1.0000x reference)
import dataclasses
import functools
import math

import numpy as np
import jax
import jax.numpy as jnp
from jax import lax
from jax.experimental import pallas as pl
from jax.experimental.pallas import tpu as pltpu
from jax.experimental.pallas import tpu_sc as plsc

_BOUND = 0.9
_BIN = 0.001
_KW = 0.02
_ND = 16
_EPS = 0.001
_NT = 1000
_EPS_SQ = _EPS * _EPS
_S = 1.0 / (_ND - 1) + _EPS
_N = 1000000

def _conv_consts():
    slope = 0.5 * _KW / _BIN
    half = np.arange(1.0, 0.0, -1.0 / slope)
    k = np.concatenate([half[::-1], half[1:]])
    k = (k / k.sum()).astype(np.float32)
    ones = np.ones((_NT,), np.float32)
    conv_ones = np.convolve(ones, k, mode="same").astype(np.float32)
    return k, conv_ones

_KVEC, _CONV_ONES = _conv_consts()
_KLEN = _KVEC.shape[0]
_PAD = (_KLEN - 1) // 2

_TBL = (_ND + 2) * _NT

def _tc_table_body(m_ref, co_ref, o_ref):
    m = _BOUND * jnp.tanh(m_ref[...])
    z = jnp.zeros((_ND, _PAD), jnp.float32)
    mp = jnp.concatenate([z, m, z], axis=1)
    acc = _KVEC[0] * lax.slice(mp, (0, 0), (_ND, _NT))
    for k in range(1, _KLEN):
        acc = acc + _KVEC[k] * lax.slice(mp, (0, k), (_ND, _NT + k))
    sm = acc / co_ref[...]
    sm = sm - jnp.mean(sm, axis=1, keepdims=True)
    zrow = jnp.zeros((1, _NT), jnp.float32)
    o_ref[...] = jnp.concatenate([zrow, sm, zrow], axis=0)


def _tc_table(motion):
    return pl.pallas_call(
        _tc_table_body,
        out_shape=jax.ShapeDtypeStruct((_ND + 2, _NT), jnp.float32),
    )(motion, jnp.asarray(_CONV_ONES)[None, :])


_CH = 4096
_NFULL = _N // _CH
_TAILN = _N - _NFULL * _CH
_MAXCH = -(-_NFULL // 32)

_C15 = np.float32(1.0) / np.float32(15.0)
_F_S = np.float32(_S)
_F_EPS = np.float32(_EPS)
_F_EPS_SQ = np.float32(_EPS_SQ)
_CP2_OFF = _C15 - _F_EPS
_INV_BIN = np.float32(1.0) / np.float32(_BIN)
_DEN0 = _F_EPS_SQ + (_F_S + _F_EPS)
_INV_DEN0 = np.float32(1.0) / _DEN0
_C15_SC = _C15 * _INV_DEN0
_S_SC = _F_S * _INV_DEN0
_EPS_SC = _F_EPS * _INV_DEN0
_CP2_OFF_SC = _CP2_OFF * _INV_DEN0


def _sc_compute_vec(tbl_v, t_v, d_v, o_v, i):
    t = t_v[pl.ds(i, 16)]
    d = d_v[pl.ds(i, 16)]
    bin_ = (t * _INV_BIN).astype(jnp.int32)
    u = d * np.float32(15.0)
    d0 = u.astype(jnp.int32)
    frac = u - d0.astype(jnp.float32)
    f15 = frac * _C15_SC
    c0 = _S_SC - f15
    cp = _EPS_SC + f15
    cm = jnp.maximum(_EPS_SC - f15, np.float32(0.0))
    cp2 = jnp.maximum(f15 - _CP2_OFF_SC, np.float32(0.0))
    base = d0 * 1000 + bin_
    gm = plsc.load_gather(tbl_v, [base])
    g0 = plsc.load_gather(tbl_v.at[pl.ds(1000, 17000)], [base])
    gp = plsc.load_gather(tbl_v.at[pl.ds(2000, 16000)], [base])
    gp2 = plsc.load_gather(tbl_v.at[pl.ds(3000, 15000)], [base])
    o_v[pl.ds(i, 16)] = (c0 * g0 + cp * gp) + (cm * gm + cp2 * gp2)


def _sc_pred(table_flat, times, depths):
    mesh = plsc.VectorSubcoreMesh(core_axis_name="c", subcore_axis_name="s")
    cp = pltpu.CompilerParams()
    if "needs_layout_passes" in pltpu.CompilerParams.__dataclass_fields__:
        cp = dataclasses.replace(cp, needs_layout_passes=False)

    @functools.partial(
        pl.kernel,
        out_type=jax.ShapeDtypeStruct((_N,), jnp.float32),
        mesh=mesh,
        compiler_params=cp,
        scratch_types=[
            pltpu.VMEM((_TBL,), jnp.float32),
            pltpu.VMEM((_CH,), jnp.float32),
            pltpu.VMEM((_CH,), jnp.float32),
            pltpu.VMEM((_CH,), jnp.float32),
            pltpu.VMEM((_CH,), jnp.float32),
            pltpu.VMEM((_CH,), jnp.float32),
            pltpu.VMEM((_CH,), jnp.float32),
            pltpu.SemaphoreType.DMA,
            pltpu.SemaphoreType.DMA,
            pltpu.SemaphoreType.DMA,
            pltpu.SemaphoreType.DMA,
            pltpu.SemaphoreType.DMA,
            pltpu.SemaphoreType.DMA,
            pltpu.SemaphoreType.DMA,
        ],
    )
    def k(tbl_hbm, t_hbm, d_hbm, o_hbm, tbl_v,
          t0, d0, o0, t1, d1, o1, ts0, ds0, os0, ts1, ds1, os1, tbsem):
        w = lax.axis_index("s") * 2 + lax.axis_index("c")
        tbl_cp = pltpu.make_async_copy(tbl_hbm, tbl_v, tbsem)
        tbl_cp.start()
        n_c = jnp.where(w < _NFULL % 32, _NFULL // 32 + 1, _NFULL // 32)

        def cbase(ci):
            return (ci * 32 + w) * _CH

        def in_start(ci, tb, db, tsem, dsem):
            b = cbase(ci)
            pltpu.make_async_copy(t_hbm.at[pl.ds(b, _CH)], tb, tsem).start()
            pltpu.make_async_copy(d_hbm.at[pl.ds(b, _CH)], db, dsem).start()

        def in_wait(ci, tb, db, tsem, dsem):
            b = cbase(ci)
            pltpu.make_async_copy(t_hbm.at[pl.ds(b, _CH)], tb, tsem).wait()
            pltpu.make_async_copy(d_hbm.at[pl.ds(b, _CH)], db, dsem).wait()

        def out_wait(ci, ob, osem):
            b = cbase(ci)
            pltpu.make_async_copy(ob, o_hbm.at[pl.ds(b, _CH)], osem).wait()

        in_start(0, t0, d0, ts0, ds0)
        in_start(1, t1, d1, ts1, ds1)
        tbl_cp.wait()

        def step(ci, tb, db, ob, tsem, dsem, osem):
            @pl.when(ci < n_c)
            def _():
                in_wait(ci, tb, db, tsem, dsem)

                @pl.when(ci >= 2)
                def _():
                    out_wait(ci - 2, ob, osem)

                @plsc.parallel_loop(0, _CH, step=16, unroll=16)
                def _(i):
                    _sc_compute_vec(tbl_v, tb, db, ob, i)

                pltpu.make_async_copy(
                    ob, o_hbm.at[pl.ds(cbase(ci), _CH)], osem).start()

                @pl.when(ci + 2 < n_c)
                def _():
                    in_start(ci + 2, tb, db, tsem, dsem)

        @pl.loop(0, _MAXCH, step=2)
        def _(ci0):
            step(ci0, t0, d0, o0, ts0, ds0, os0)
            step(ci0 + 1, t1, d1, o1, ts1, ds1, os1)

        _q = _NFULL // 32
        out_wait(_q - 1, o0, os0)
        out_wait(jnp.where(n_c == _q + 1, _q, _q - 2), o1, os1)

        @pl.when(w == 31)
        def _():
            b = _NFULL * _CH
            pltpu.sync_copy(t_hbm.at[pl.ds(b, _TAILN)], t0.at[pl.ds(0, _TAILN)])
            pltpu.sync_copy(d_hbm.at[pl.ds(b, _TAILN)], d0.at[pl.ds(0, _TAILN)])

            @plsc.parallel_loop(0, _TAILN, step=16, unroll=16)
            def _(i):
                _sc_compute_vec(tbl_v, t0, d0, o0, i)

            pltpu.sync_copy(o0.at[pl.ds(0, _TAILN)], o_hbm.at[pl.ds(b, _TAILN)])

    return k(table_flat, times, depths)


def kernel(times, depths, motion):
    table = _tc_table(motion).reshape((_TBL,))
    return _sc_pred(table, times, depths)


# speedup vs baseline: 1.3043x; 1.3043x over previous
import dataclasses
import functools
import math

import numpy as np
import jax
import jax.numpy as jnp
from jax import lax
from jax.experimental import pallas as pl
from jax.experimental.pallas import tpu as pltpu
from jax.experimental.pallas import tpu_sc as plsc

_BOUND = 0.9
_BIN = 0.001
_KW = 0.02
_ND = 16
_EPS = 0.001
_NT = 1000
_EPS_SQ = _EPS * _EPS
_S = 1.0 / (_ND - 1) + _EPS
_N = 1000000

def _conv_consts():
    slope = 0.5 * _KW / _BIN
    half = np.arange(1.0, 0.0, -1.0 / slope)
    k = np.concatenate([half[::-1], half[1:]])
    k = (k / k.sum()).astype(np.float32)
    ones = np.ones((_NT,), np.float32)
    conv_ones = np.convolve(ones, k, mode="same").astype(np.float32)
    return k, conv_ones

_KVEC, _CONV_ONES = _conv_consts()
_KLEN = _KVEC.shape[0]
_PAD = (_KLEN - 1) // 2

_TBL = (_ND + 2) * _NT

def _tc_table_body(m_ref, co_ref, o_ref):
    m = _BOUND * jnp.tanh(m_ref[...])
    z = jnp.zeros((_ND, _PAD), jnp.float32)
    mp = jnp.concatenate([z, m, z], axis=1)
    acc = _KVEC[0] * lax.slice(mp, (0, 0), (_ND, _NT))
    for k in range(1, _KLEN):
        acc = acc + _KVEC[k] * lax.slice(mp, (0, k), (_ND, _NT + k))
    sm = acc / co_ref[...]
    sm = sm - jnp.mean(sm, axis=1, keepdims=True)
    zrow = jnp.zeros((1, _NT), jnp.float32)
    o_ref[...] = jnp.concatenate([zrow, sm, zrow], axis=0)


def _tc_table(motion):
    return pl.pallas_call(
        _tc_table_body,
        out_shape=jax.ShapeDtypeStruct((_ND + 2, _NT), jnp.float32),
    )(motion, jnp.asarray(_CONV_ONES)[None, :])


_CH = 4096
_NFULL = _N // _CH
_TAILN = _N - _NFULL * _CH
_MAXCH = -(-_NFULL // 32)

_C15 = np.float32(1.0) / np.float32(15.0)
_F_S = np.float32(_S)
_F_EPS = np.float32(_EPS)
_F_EPS_SQ = np.float32(_EPS_SQ)
_CP2_OFF = _C15 - _F_EPS
_INV_BIN = np.float32(1.0) / np.float32(_BIN)
_DEN0 = _F_EPS_SQ + (_F_S + _F_EPS)
_INV_DEN0 = np.float32(1.0) / _DEN0
_C15_SC = _C15 * _INV_DEN0
_S_SC = _F_S * _INV_DEN0
_EPS_SC = _F_EPS * _INV_DEN0
_CP2_OFF_SC = _CP2_OFF * _INV_DEN0


def _sc_compute_vec(tbl_v, t_v, d_v, o_v, i):
    t = t_v[pl.ds(i, 16)]
    d = d_v[pl.ds(i, 16)]
    bin_ = (t * _INV_BIN).astype(jnp.int32)
    u = d * np.float32(15.0)
    d0 = u.astype(jnp.int32)
    frac = u - d0.astype(jnp.float32)
    f15 = frac * _C15_SC
    c0 = _S_SC - f15
    cp = _EPS_SC + f15
    cm = jnp.maximum(_EPS_SC - f15, np.float32(0.0))
    cp2 = jnp.maximum(f15 - _CP2_OFF_SC, np.float32(0.0))
    base = d0 * 1000 + bin_
    gm = plsc.load_gather(tbl_v, [base])
    g0 = plsc.load_gather(tbl_v.at[pl.ds(1000, 17000)], [base])
    gp = plsc.load_gather(tbl_v.at[pl.ds(2000, 16000)], [base])
    gp2 = plsc.load_gather(tbl_v.at[pl.ds(3000, 15000)], [base])
    o_v[pl.ds(i, 16)] = (c0 * g0 + cp * gp) + (cm * gm + cp2 * gp2)


def _sc_pred(table_flat, times, depths):
    mesh = plsc.VectorSubcoreMesh(core_axis_name="c", subcore_axis_name="s")
    cp = pltpu.CompilerParams()
    if "needs_layout_passes" in pltpu.CompilerParams.__dataclass_fields__:
        cp = dataclasses.replace(cp, needs_layout_passes=False)

    @functools.partial(
        pl.kernel,
        out_type=jax.ShapeDtypeStruct((_N,), jnp.float32),
        mesh=mesh,
        compiler_params=cp,
        scratch_types=[
            pltpu.VMEM((_TBL,), jnp.float32),
            pltpu.VMEM((_CH,), jnp.float32),
            pltpu.VMEM((_CH,), jnp.float32),
            pltpu.VMEM((_CH,), jnp.float32),
            pltpu.VMEM((_CH,), jnp.float32),
            pltpu.VMEM((_CH,), jnp.float32),
            pltpu.VMEM((_CH,), jnp.float32),
            pltpu.SemaphoreType.DMA,
            pltpu.SemaphoreType.DMA,
            pltpu.SemaphoreType.DMA,
            pltpu.SemaphoreType.DMA,
            pltpu.SemaphoreType.DMA,
            pltpu.SemaphoreType.DMA,
            pltpu.SemaphoreType.DMA,
        ],
    )
    def k(tbl_hbm, t_hbm, d_hbm, o_hbm, tbl_v,
          t0, d0, o0, t1, d1, o1, ts0, ds0, os0, ts1, ds1, os1, tbsem):
        w = lax.axis_index("s") * 2 + lax.axis_index("c")
        tbl_cp = pltpu.make_async_copy(tbl_hbm, tbl_v, tbsem)
        tbl_cp.start()
        n_c = jnp.where(w < _NFULL % 32, _NFULL // 32 + 1, _NFULL // 32)

        def cbase(ci):
            return (ci * 32 + w) * _CH

        def in_start(ci, tb, db, tsem, dsem):
            b = cbase(ci)
            pltpu.make_async_copy(t_hbm.at[pl.ds(b, _CH)], tb, tsem).start()
            pltpu.make_async_copy(d_hbm.at[pl.ds(b, _CH)], db, dsem).start()

        def in_wait(ci, tb, db, tsem, dsem):
            b = cbase(ci)
            pltpu.make_async_copy(t_hbm.at[pl.ds(b, _CH)], tb, tsem).wait()
            pltpu.make_async_copy(d_hbm.at[pl.ds(b, _CH)], db, dsem).wait()

        def out_wait(ci, ob, osem):
            b = cbase(ci)
            pltpu.make_async_copy(ob, o_hbm.at[pl.ds(b, _CH)], osem).wait()

        in_start(0, t0, d0, ts0, ds0)
        in_start(1, t1, d1, ts1, ds1)
        tbl_cp.wait()

        def step(ci, tb, db, ob, tsem, dsem, osem):
            @pl.when(ci < n_c)
            def _():
                in_wait(ci, tb, db, tsem, dsem)

                @pl.when(ci >= 2)
                def _():
                    out_wait(ci - 2, ob, osem)

                @plsc.parallel_loop(0, _CH, step=16, unroll=6)
                def _(i):
                    _sc_compute_vec(tbl_v, tb, db, ob, i)

                pltpu.make_async_copy(
                    ob, o_hbm.at[pl.ds(cbase(ci), _CH)], osem).start()

                @pl.when(ci + 2 < n_c)
                def _():
                    in_start(ci + 2, tb, db, tsem, dsem)

        @pl.loop(0, _MAXCH, step=2)
        def _(ci0):
            step(ci0, t0, d0, o0, ts0, ds0, os0)
            step(ci0 + 1, t1, d1, o1, ts1, ds1, os1)

        _q = _NFULL // 32
        out_wait(_q - 1, o0, os0)
        out_wait(jnp.where(n_c == _q + 1, _q, _q - 2), o1, os1)

        @pl.when(w == 31)
        def _():
            b = _NFULL * _CH
            pltpu.sync_copy(t_hbm.at[pl.ds(b, _TAILN)], t0.at[pl.ds(0, _TAILN)])
            pltpu.sync_copy(d_hbm.at[pl.ds(b, _TAILN)], d0.at[pl.ds(0, _TAILN)])

            @plsc.parallel_loop(0, _TAILN, step=16, unroll=6)
            def _(i):
                _sc_compute_vec(tbl_v, t0, d0, o0, i)

            pltpu.sync_copy(o0.at[pl.ds(0, _TAILN)], o_hbm.at[pl.ds(b, _TAILN)])

    return k(table_flat, times, depths)


def kernel(times, depths, motion):
    table = _tc_table(motion).reshape((_TBL,))
    return _sc_pred(table, times, depths)


# speedup vs baseline: 1.3155x; 1.0086x over previous
import dataclasses
import functools
import math

import numpy as np
import jax
import jax.numpy as jnp
from jax import lax
from jax.experimental import pallas as pl
from jax.experimental.pallas import tpu as pltpu
from jax.experimental.pallas import tpu_sc as plsc

_BOUND = 0.9
_BIN = 0.001
_KW = 0.02
_ND = 16
_EPS = 0.001
_NT = 1000
_EPS_SQ = _EPS * _EPS
_S = 1.0 / (_ND - 1) + _EPS
_N = 1000000

def _conv_consts():
    slope = 0.5 * _KW / _BIN
    half = np.arange(1.0, 0.0, -1.0 / slope)
    k = np.concatenate([half[::-1], half[1:]])
    k = (k / k.sum()).astype(np.float32)
    ones = np.ones((_NT,), np.float32)
    conv_ones = np.convolve(ones, k, mode="same").astype(np.float32)
    return k, conv_ones

_KVEC, _CONV_ONES = _conv_consts()
_KLEN = _KVEC.shape[0]
_PAD = (_KLEN - 1) // 2

_TBL = (_ND + 2) * _NT

def _tc_table_body(m_ref, co_ref, o_ref):
    m = _BOUND * jnp.tanh(m_ref[...])
    z = jnp.zeros((_ND, _PAD), jnp.float32)
    mp = jnp.concatenate([z, m, z], axis=1)
    acc = _KVEC[0] * lax.slice(mp, (0, 0), (_ND, _NT))
    for k in range(1, _KLEN):
        acc = acc + _KVEC[k] * lax.slice(mp, (0, k), (_ND, _NT + k))
    sm = acc / co_ref[...]
    sm = sm - jnp.mean(sm, axis=1, keepdims=True)
    zrow = jnp.zeros((1, _NT), jnp.float32)
    o_ref[...] = jnp.concatenate([zrow, sm, zrow], axis=0)


def _tc_table(motion):
    return pl.pallas_call(
        _tc_table_body,
        out_shape=jax.ShapeDtypeStruct((_ND + 2, _NT), jnp.float32),
    )(motion, jnp.asarray(_CONV_ONES)[None, :])


_CH = 4096
_NFULL = _N // _CH
_TAILN = _N - _NFULL * _CH
_MAXCH = -(-_NFULL // 32)

_C15 = np.float32(1.0) / np.float32(15.0)
_F_S = np.float32(_S)
_F_EPS = np.float32(_EPS)
_F_EPS_SQ = np.float32(_EPS_SQ)
_CP2_OFF = _C15 - _F_EPS
_INV_BIN = np.float32(1.0) / np.float32(_BIN)
_DEN0 = _F_EPS_SQ + (_F_S + _F_EPS)
_INV_DEN0 = np.float32(1.0) / _DEN0
_C15_SC = _C15 * _INV_DEN0
_S_SC = _F_S * _INV_DEN0
_EPS_SC = _F_EPS * _INV_DEN0
_CP2_OFF_SC = _CP2_OFF * _INV_DEN0


def _sc_compute_vec(tbl_v, t_v, d_v, o_v, i):
    t = t_v[pl.ds(i, 16)]
    d = d_v[pl.ds(i, 16)]
    bin_ = (t * _INV_BIN).astype(jnp.int32)
    u = d * np.float32(15.0)
    d0 = u.astype(jnp.int32)
    frac = u - d0.astype(jnp.float32)
    f15 = frac * _C15_SC
    c0 = _S_SC - f15
    cp = _EPS_SC + f15
    cm = jnp.maximum(_EPS_SC - f15, np.float32(0.0))
    cp2 = jnp.maximum(f15 - _CP2_OFF_SC, np.float32(0.0))
    base = d0 * 1000 + bin_
    gm = plsc.load_gather(tbl_v, [base])
    g0 = plsc.load_gather(tbl_v.at[pl.ds(1000, 17000)], [base])
    gp = plsc.load_gather(tbl_v.at[pl.ds(2000, 16000)], [base])
    gp2 = plsc.load_gather(tbl_v.at[pl.ds(3000, 15000)], [base])
    o_v[pl.ds(i, 16)] = (c0 * g0 + cp * gp) + (cm * gm + cp2 * gp2)


def _sc_pred(table_flat, times, depths):
    mesh = plsc.VectorSubcoreMesh(core_axis_name="c", subcore_axis_name="s")
    cp = pltpu.CompilerParams()
    if "needs_layout_passes" in pltpu.CompilerParams.__dataclass_fields__:
        cp = dataclasses.replace(cp, needs_layout_passes=False)

    @functools.partial(
        pl.kernel,
        out_type=jax.ShapeDtypeStruct((_N,), jnp.float32),
        mesh=mesh,
        compiler_params=cp,
        scratch_types=[
            pltpu.VMEM((_TBL,), jnp.float32),
            pltpu.VMEM((_CH,), jnp.float32),
            pltpu.VMEM((_CH,), jnp.float32),
            pltpu.VMEM((_CH,), jnp.float32),
            pltpu.VMEM((_CH,), jnp.float32),
            pltpu.VMEM((_CH,), jnp.float32),
            pltpu.VMEM((_CH,), jnp.float32),
            pltpu.SemaphoreType.DMA,
            pltpu.SemaphoreType.DMA,
            pltpu.SemaphoreType.DMA,
            pltpu.SemaphoreType.DMA,
            pltpu.SemaphoreType.DMA,
            pltpu.SemaphoreType.DMA,
            pltpu.SemaphoreType.DMA,
        ],
    )
    def k(tbl_hbm, t_hbm, d_hbm, o_hbm, tbl_v,
          t0, d0, o0, t1, d1, o1, ts0, ds0, os0, ts1, ds1, os1, tbsem):
        w = lax.axis_index("s") * 2 + lax.axis_index("c")
        tbl_cp = pltpu.make_async_copy(tbl_hbm, tbl_v, tbsem)
        tbl_cp.start()
        n_c = jnp.where(w < _NFULL % 32, _NFULL // 32 + 1, _NFULL // 32)

        def cbase(ci):
            return (ci * 32 + w) * _CH

        def in_start(ci, tb, db, tsem, dsem):
            b = cbase(ci)
            pltpu.make_async_copy(t_hbm.at[pl.ds(b, _CH)], tb, tsem).start()
            pltpu.make_async_copy(d_hbm.at[pl.ds(b, _CH)], db, dsem).start()

        def in_wait(ci, tb, db, tsem, dsem):
            b = cbase(ci)
            pltpu.make_async_copy(t_hbm.at[pl.ds(b, _CH)], tb, tsem).wait()
            pltpu.make_async_copy(d_hbm.at[pl.ds(b, _CH)], db, dsem).wait()

        def out_wait(ci, ob, osem):
            b = cbase(ci)
            pltpu.make_async_copy(ob, o_hbm.at[pl.ds(b, _CH)], osem).wait()

        in_start(0, t0, d0, ts0, ds0)
        in_start(1, t1, d1, ts1, ds1)
        tbl_cp.wait()

        def step(ci, tb, db, ob, tsem, dsem, osem):
            @pl.when(ci < n_c)
            def _():
                in_wait(ci, tb, db, tsem, dsem)

                @pl.when(ci >= 2)
                def _():
                    out_wait(ci - 2, ob, osem)

                @plsc.parallel_loop(0, _CH, step=16, unroll=8)
                def _(i):
                    _sc_compute_vec(tbl_v, tb, db, ob, i)

                pltpu.make_async_copy(
                    ob, o_hbm.at[pl.ds(cbase(ci), _CH)], osem).start()

                @pl.when(ci + 2 < n_c)
                def _():
                    in_start(ci + 2, tb, db, tsem, dsem)

        @pl.loop(0, _MAXCH, step=2)
        def _(ci0):
            step(ci0, t0, d0, o0, ts0, ds0, os0)
            step(ci0 + 1, t1, d1, o1, ts1, ds1, os1)

        _q = _NFULL // 32
        out_wait(_q - 1, o0, os0)
        out_wait(jnp.where(n_c == _q + 1, _q, _q - 2), o1, os1)

        @pl.when(w == 31)
        def _():
            b = _NFULL * _CH
            pltpu.sync_copy(t_hbm.at[pl.ds(b, _TAILN)], t0.at[pl.ds(0, _TAILN)])
            pltpu.sync_copy(d_hbm.at[pl.ds(b, _TAILN)], d0.at[pl.ds(0, _TAILN)])

            @plsc.parallel_loop(0, _TAILN, step=16, unroll=8)
            def _(i):
                _sc_compute_vec(tbl_v, t0, d0, o0, i)

            pltpu.sync_copy(o0.at[pl.ds(0, _TAILN)], o_hbm.at[pl.ds(b, _TAILN)])

    return k(table_flat, times, depths)


def kernel(times, depths, motion):
    table = _tc_table(motion).reshape((_TBL,))
    return _sc_pred(table, times, depths)


# speedup vs baseline: 1.3181x; 1.0020x over previous
import dataclasses
import functools

import numpy as np
import jax
import jax.numpy as jnp
from jax import lax
from jax.experimental import pallas as pl
from jax.experimental.pallas import tpu as pltpu
from jax.experimental.pallas import tpu_sc as plsc

_BOUND = 0.9
_BIN = 0.001
_KW = 0.02
_ND = 16
_EPS = 0.001
_NT = 1000
_EPS_SQ = _EPS * _EPS
_S = 1.0 / (_ND - 1) + _EPS
_N = 1000000

def _conv_consts():
    slope = 0.5 * _KW / _BIN
    half = np.arange(1.0, 0.0, -1.0 / slope)
    k = np.concatenate([half[::-1], half[1:]])
    k = (k / k.sum()).astype(np.float32)
    ones = np.ones((_NT,), np.float32)
    conv_ones = np.convolve(ones, k, mode="same").astype(np.float32)
    return k, conv_ones

_KVEC, _CONV_ONES = _conv_consts()
_KLEN = _KVEC.shape[0]
_PAD = (_KLEN - 1) // 2

_TBL = (_ND + 2) * _NT

def _tc_table_body(m_ref, co_ref, o_ref):
    m = _BOUND * jnp.tanh(m_ref[...])
    z = jnp.zeros((_ND, _PAD), jnp.float32)
    mp = jnp.concatenate([z, m, z], axis=1)
    acc = _KVEC[0] * lax.slice(mp, (0, 0), (_ND, _NT))
    for k in range(1, _KLEN):
        acc = acc + _KVEC[k] * lax.slice(mp, (0, k), (_ND, _NT + k))
    sm = acc / co_ref[...]
    sm = sm - jnp.mean(sm, axis=1, keepdims=True)
    zrow = jnp.zeros((1, _NT), jnp.float32)
    o_ref[...] = jnp.concatenate([zrow, sm, zrow], axis=0)


def _tc_table(motion):
    return pl.pallas_call(
        _tc_table_body,
        out_shape=jax.ShapeDtypeStruct((_ND + 2, _NT), jnp.float32),
    )(motion, jnp.asarray(_CONV_ONES)[None, :])


_CH = 4096
_NFULL = _N // _CH
_TAILN = _N - _NFULL * _CH
_MAXCH = -(-_NFULL // 32)

_C15 = np.float32(1.0) / np.float32(15.0)
_F_S = np.float32(_S)
_F_EPS = np.float32(_EPS)
_F_EPS_SQ = np.float32(_EPS_SQ)
_CP2_OFF = _C15 - _F_EPS
_INV_BIN = np.float32(1.0) / np.float32(_BIN)
_DEN0 = _F_EPS_SQ + (_F_S + _F_EPS)
_INV_DEN0 = np.float32(1.0) / _DEN0
_C15_SC = _C15 * _INV_DEN0
_S_SC = _F_S * _INV_DEN0
_EPS_SC = _F_EPS * _INV_DEN0
_CP2_OFF_SC = _CP2_OFF * _INV_DEN0


def _sc_compute_vec(tbl_v, t_v, d_v, o_v, i):
    t = t_v[pl.ds(i, 16)]
    d = d_v[pl.ds(i, 16)]
    bin_ = (t * _INV_BIN).astype(jnp.int32)
    u = d * np.float32(15.0)
    d0 = u.astype(jnp.int32)
    frac = u - d0.astype(jnp.float32)
    f15 = frac * _C15_SC
    c0 = _S_SC - f15
    cp = _EPS_SC + f15
    cm = jnp.maximum(_EPS_SC - f15, np.float32(0.0))
    cp2 = jnp.maximum(f15 - _CP2_OFF_SC, np.float32(0.0))
    base = d0 * 1000 + bin_
    gm = plsc.load_gather(tbl_v, [base])
    g0 = plsc.load_gather(tbl_v.at[pl.ds(1000, 17000)], [base])
    gp = plsc.load_gather(tbl_v.at[pl.ds(2000, 16000)], [base])
    gp2 = plsc.load_gather(tbl_v.at[pl.ds(3000, 15000)], [base])
    o_v[pl.ds(i, 16)] = (c0 * g0 + cp * gp) + (cm * gm + cp2 * gp2)


def _sc_pred(table_flat, times, depths):
    mesh = plsc.VectorSubcoreMesh(core_axis_name="c", subcore_axis_name="s")
    cp = pltpu.CompilerParams()
    if "needs_layout_passes" in pltpu.CompilerParams.__dataclass_fields__:
        cp = dataclasses.replace(cp, needs_layout_passes=False)

    @functools.partial(
        pl.kernel,
        out_type=jax.ShapeDtypeStruct((_N,), jnp.float32),
        mesh=mesh,
        compiler_params=cp,
        scratch_types=[
            pltpu.VMEM((_TBL,), jnp.float32),
            pltpu.VMEM((_CH,), jnp.float32),
            pltpu.VMEM((_CH,), jnp.float32),
            pltpu.VMEM((_CH,), jnp.float32),
            pltpu.VMEM((_CH,), jnp.float32),
            pltpu.VMEM((_CH,), jnp.float32),
            pltpu.VMEM((_CH,), jnp.float32),
            pltpu.SemaphoreType.DMA,
            pltpu.SemaphoreType.DMA,
            pltpu.SemaphoreType.DMA,
            pltpu.SemaphoreType.DMA,
            pltpu.SemaphoreType.DMA,
            pltpu.SemaphoreType.DMA,
            pltpu.SemaphoreType.DMA,
        ],
    )
    def k(tbl_hbm, t_hbm, d_hbm, o_hbm, tbl_v,
          t0, d0, o0, t1, d1, o1, ts0, ds0, os0, ts1, ds1, os1, tbsem):
        w = lax.axis_index("s") * 2 + lax.axis_index("c")
        tbl_cp = pltpu.make_async_copy(tbl_hbm, tbl_v, tbsem)
        tbl_cp.start()
        n_c = jnp.where(w < _NFULL % 32, _NFULL // 32 + 1, _NFULL // 32)

        def cbase(ci):
            return (ci * 32 + w) * _CH

        def in_start(ci, tb, db, tsem, dsem):
            b = cbase(ci)
            pltpu.make_async_copy(t_hbm.at[pl.ds(b, _CH)], tb, tsem).start()
            pltpu.make_async_copy(d_hbm.at[pl.ds(b, _CH)], db, dsem).start()

        def in_wait(ci, tb, db, tsem, dsem):
            b = cbase(ci)
            pltpu.make_async_copy(t_hbm.at[pl.ds(b, _CH)], tb, tsem).wait()
            pltpu.make_async_copy(d_hbm.at[pl.ds(b, _CH)], db, dsem).wait()

        def out_wait(ci, ob, osem):
            b = cbase(ci)
            pltpu.make_async_copy(ob, o_hbm.at[pl.ds(b, _CH)], osem).wait()

        in_start(0, t0, d0, ts0, ds0)
        in_start(1, t1, d1, ts1, ds1)
        tbl_cp.wait()

        def step(ci, tb, db, ob, tsem, dsem, osem):
            @pl.when(ci < n_c)
            def _():
                in_wait(ci, tb, db, tsem, dsem)

                @pl.when(ci >= 2)
                def _():
                    out_wait(ci - 2, ob, osem)

                @plsc.parallel_loop(0, _CH, step=16, unroll=8)
                def _(i):
                    _sc_compute_vec(tbl_v, tb, db, ob, i)

                pltpu.make_async_copy(
                    ob, o_hbm.at[pl.ds(cbase(ci), _CH)], osem).start()

                @pl.when(ci + 2 < n_c)
                def _():
                    in_start(ci + 2, tb, db, tsem, dsem)

        @pl.loop(0, _MAXCH, step=2)
        def _(ci0):
            step(ci0, t0, d0, o0, ts0, ds0, os0)
            step(ci0 + 1, t1, d1, o1, ts1, ds1, os1)

        _q = _NFULL // 32
        out_wait(_q - 1, o0, os0)
        out_wait(jnp.where(n_c == _q + 1, _q, _q - 2), o1, os1)

        @pl.when(w == 31)
        def _():
            b = _NFULL * _CH
            pltpu.sync_copy(t_hbm.at[pl.ds(b, _TAILN)], t0.at[pl.ds(0, _TAILN)])
            pltpu.sync_copy(d_hbm.at[pl.ds(b, _TAILN)], d0.at[pl.ds(0, _TAILN)])

            @plsc.parallel_loop(0, _TAILN, step=16, unroll=8)
            def _(i):
                _sc_compute_vec(tbl_v, t0, d0, o0, i)

            pltpu.sync_copy(o0.at[pl.ds(0, _TAILN)], o_hbm.at[pl.ds(b, _TAILN)])

    return k(table_flat, times, depths)


def kernel(times, depths, motion):
    table = _tc_table(motion).reshape((_TBL,))
    return _sc_pred(table, times, depths)
